# Initial kernel scaffold; baseline (speedup 1.0000x reference)
import jax
import jax.numpy as jnp
from jax.experimental import pallas as pl


def kernel(x, W1, b1, W2, b2, species_emb, Wq, bq, Wk, bk, Wv, bv, Wo, bo, ln_g, ln_b, Wh, bh, r_intr):
    raise NotImplementedError("write your pallas kernel here")


# speedup vs baseline: 20.1685x; 20.1685x over previous
import jax
import jax.numpy as jnp
from functools import partial
from jax.experimental import pallas as pl

def _dot(a, b, dn):
    return jax.lax.dot_general(a.astype(jnp.bfloat16), b.astype(jnp.bfloat16),
                               dn, preferred_element_type=jnp.float32)

_B, _T, _N = 4, 64, 256
_D, _H, _L, _TOPK = 32, 2, 2, 3
_DH = _D // _H
_CHUNK = 8


def _ln(x2d, g, b):
    m = jnp.mean(x2d, axis=-1, keepdims=True)
    v = jnp.mean((x2d - m) ** 2, axis=-1, keepdims=True)
    return (x2d - m) * jax.lax.rsqrt(v + 1e-5) * g + b


def _fused_kernel(x_ref, W1_ref, b1_ref, W2_ref, b2_ref, emb_ref,
                  Wq_ref, bq_ref, Wk_ref, bk_ref, Wv_ref, bv_ref,
                  Wo_ref, bo_ref, lng_ref, lnb_ref, Wh_ref, bh_ref,
                  r_ref, out_ref):
    C = x_ref.shape[0]
    xb = x_ref[...]
    xl = jnp.log(jnp.maximum(xb, 1e-6))

    W1 = W1_ref[...].astype(jnp.bfloat16).astype(jnp.float32)
    xb16 = xb.astype(jnp.bfloat16).astype(jnp.float32)
    xl16 = xl.astype(jnp.bfloat16).astype(jnp.float32)
    h = (xb16[..., None] * W1[:, 0][None, None, :]
         + xl16[..., None] * W1[:, 1][None, None, :]
         + b1_ref[...][None, :, :])
    h = h * 0.5 * (1.0 + jax.lax.erf(h * (2.0 ** -0.5)))
    h2 = _dot(h.reshape(C * _N, _D), W2_ref[...],
                             (((1,), (1,)), ((), ())))
    hf = ((h2 + b2_ref[...]).reshape(C, _N, _D)
          + emb_ref[...][None, :, :]).reshape(C * _N, _D)

    for l in range(_L):
        q = _dot(hf, Wq_ref[l], (((1,), (1,)), ((), ()))) + bq_ref[l][None, :]
        k = _dot(hf, Wk_ref[l], (((1,), (1,)), ((), ()))) + bk_ref[l][None, :]
        v = _dot(hf, Wv_ref[l], (((1,), (1,)), ((), ()))) + bv_ref[l][None, :]
        q3 = q.reshape(C, _N, _D)
        k3 = k.reshape(C, _N, _D)
        v3 = v.reshape(C, _N, _D)
        heads = []
        for hd in range(_H):
            sl = slice(hd * _DH, (hd + 1) * _DH)
            qh, kh, vh = q3[:, :, sl], k3[:, :, sl], v3[:, :, sl]
            s = _dot(qh, kh, (((2,), (2,)), ((0,), (0,)))) * (_DH ** -0.5)
            iota = jax.lax.broadcasted_iota(jnp.int32, s.shape, 2)
            scur = s
            m1 = None
            ws, ohs = [], []
            for t in range(_TOPK):
                mv = jnp.max(scur, axis=2, keepdims=True)
                idx = jnp.min(jnp.where(scur == mv, iota, _N), axis=2, keepdims=True)
                oh = iota == idx
                if t == 0:
                    m1 = mv
                ws.append(jnp.exp(mv - m1))
                ohs.append(oh)
                if t < _TOPK - 1:
                    scur = jnp.where(oh, -jnp.inf, scur)
            z = ws[0] + ws[1] + ws[2]
            ex = (jnp.where(ohs[0], ws[0], 0.0)
                  + jnp.where(ohs[1], ws[1], 0.0)
                  + jnp.where(ohs[2], ws[2], 0.0))
            heads.append(_dot(ex, vh, (((2,), (1,)), ((0,), (0,)))) / z)
        o3 = jnp.concatenate(heads, axis=-1)
        o = _dot(o3.reshape(C * _N, _D), Wo_ref[l],
                                (((1,), (1,)), ((), ()))) + bo_ref[l][None, :]
        hf = _ln(hf + o, lng_ref[l][None, :], lnb_ref[l][None, :])

    hf16 = hf.astype(jnp.bfloat16).astype(jnp.float32)
    Wh16 = Wh_ref[...].astype(jnp.bfloat16).astype(jnp.float32)
    out = jnp.sum(hf16.reshape(C, _N, _D) * Wh16[None, :, :], axis=2)
    out_ref[...] = out + bh_ref[0, 0] + r_ref[...]


def kernel(x, W1, b1, W2, b2, species_emb, Wq, bq, Wk, bk, Wv, bv, Wo, bo,
           ln_g, ln_b, Wh, bh, r_intr):
    BT = _B * _T
    x2 = x.reshape(BT, _N)
    b1r = b1.reshape(1, _D)
    b2r = b2.reshape(1, _D)
    bhr = bh.reshape(1, 1)
    rr = r_intr.reshape(1, _N)

    full = lambda a: pl.BlockSpec(a.shape, lambda i: (0,) * a.ndim)
    grid = (BT // _CHUNK,)
    out = pl.pallas_call(
        _fused_kernel,
        grid=grid,
        in_specs=[
            pl.BlockSpec((_CHUNK, _N), lambda i: (i, 0)),
            full(W1), full(b1r), full(W2), full(b2r), full(species_emb),
            full(Wq), full(bq), full(Wk), full(bk), full(Wv), full(bv),
            full(Wo), full(bo), full(ln_g), full(ln_b), full(Wh), full(bhr),
            pl.BlockSpec((1, _N), lambda i: (0, 0)),
        ],
        out_specs=pl.BlockSpec((_CHUNK, _N), lambda i: (i, 0)),
        out_shape=jax.ShapeDtypeStruct((BT, _N), jnp.float32),
    )(x2, W1, b1r, W2, b2r, species_emb, Wq, bq, Wk, bk, Wv, bv, Wo, bo,
      ln_g, ln_b, Wh, bhr, rr)
    return out.reshape(_B, _T, _N)
